# baseline (speedup 1.0000x reference)
import functools

import jax
import jax.numpy as jnp
from jax import lax
from jax.experimental import pallas as pl
from jax.experimental.pallas import tpu as pltpu
from jax.experimental.pallas import tpu_sc as plsc

N = 10000
D = 128
E = 320000
CHUNK = 128
NC = 2
NS = 16
NW = NC * NS
R = 80
EPAD = NW * R * CHUNK
NP = 10240
RPT = NP // NS
ZR = 32
NBANK = 4
NROW = 2

_mesh = plsc.VectorSubcoreMesh(core_axis_name="c", subcore_axis_name="s")
_f32 = jnp.float32


def _sc_scatter_body(g_hbm, src_hbm, dst_hbm, out_hbm,
                     srcb, dstb, rows, stg, acc, semi, semg, semo):
    c = lax.axis_index("c")
    s = lax.axis_index("s")
    wid = s * NC + c
    base = wid * R

    def idx_start(n, b):
        off = (base + n) * CHUNK
        pltpu.async_copy(src_hbm.at[pl.ds(off, CHUNK)], srcb[b], semi[b])
        pltpu.async_copy(dst_hbm.at[pl.ds(off, CHUNK)], dstb[b], semi[b])

    def idx_wait(n, b):
        off = (base + n) * CHUNK
        pltpu.make_async_copy(src_hbm.at[pl.ds(off, CHUNK)], srcb[b],
                              semi[b]).wait()
        pltpu.make_async_copy(dst_hbm.at[pl.ds(off, CHUNK)], dstb[b],
                              semi[b]).wait()

    def g_start(ib, rb):
        pltpu.async_copy(g_hbm.at[srcb[ib]], rows[rb], semg[rb])

    def g_wait(ib, rb):
        pltpu.make_async_copy(g_hbm.at[srcb[ib]], rows[rb], semg[rb]).wait()

    def zinit(i, carry):
        for r16 in range(16):
            for k in range(D // 16):
                stg[i * 16 + r16, pl.ds(k * 16, 16)] = jnp.zeros((16,), _f32)
        return carry

    lax.fori_loop(0, ZR // 16, zinit, 0, unroll=False)
    for k in range(RPT // ZR):
        pltpu.async_copy(stg, acc.at[pl.ds(s * RPT + k * ZR, ZR)], semo[0])

    for b in range(NBANK):
        idx_start(b, b)
    idx_wait(0, 0)
    g_start(0, 0)
    idx_wait(1, 1)
    g_start(1, 1)

    for k in range(RPT // ZR):
        pltpu.make_async_copy(stg, acc.at[pl.ds(s * RPT + k * ZR, ZR)],
                              semo[0]).wait()
    plsc.subcore_barrier()

    def step(n, i):
        rb = i % NROW
        g_wait(i, rb)
        pltpu.sync_copy(rows[rb], acc.at[dstb[i]], add=True)

        @pl.when(n + 2 < R)
        def _():
            idx_wait(n + 2, (i + 2) % NBANK)
            g_start((i + 2) % NBANK, rb)

        @pl.when(n + 4 < R)
        def _():
            idx_start(n + 4, i)

    def body(q, carry):
        for i in range(NBANK):
            step(q * NBANK + i, i)
        return carry

    lax.fori_loop(0, R // NBANK, body, 0, unroll=False)
    plsc.subcore_barrier()

    rbase = s * RPT
    for k in range(RPT // CHUNK):
        b = k % 2
        if k >= 2:
            pltpu.make_async_copy(
                rows[b], out_hbm.at[c, pl.ds(rbase + (k - 2) * CHUNK, CHUNK)],
                semo[b]).wait()
        pltpu.sync_copy(acc.at[pl.ds(rbase + k * CHUNK, CHUNK)], rows[b])
        pltpu.async_copy(
            rows[b], out_hbm.at[c, pl.ds(rbase + k * CHUNK, CHUNK)], semo[b])
    for k in (3, 4):
        b = k % 2
        pltpu.make_async_copy(
            rows[b], out_hbm.at[c, pl.ds(rbase + k * CHUNK, CHUNK)],
            semo[b]).wait()


_sc_scatter = functools.partial(
    pl.kernel,
    out_type=jax.ShapeDtypeStruct((NC, NP, D), _f32),
    mesh=_mesh,
    scratch_types=[
        [pltpu.VMEM((CHUNK,), jnp.int32) for _ in range(NBANK)],
        [pltpu.VMEM((CHUNK,), jnp.int32) for _ in range(NBANK)],
        [pltpu.VMEM((CHUNK, D), _f32) for _ in range(NROW)],
        pltpu.VMEM((ZR, D), _f32),
        pltpu.VMEM_SHARED((NP, D), _f32),
        [pltpu.SemaphoreType.DMA for _ in range(NBANK)],
        [pltpu.SemaphoreType.DMA for _ in range(NROW)],
        [pltpu.SemaphoreType.DMA for _ in range(2)],
    ],
)(_sc_scatter_body)


def _sc_deg_body(dst_hbm, out_hbm, dstb, ones, stg, acc, semi):
    c = lax.axis_index("c")
    s = lax.axis_index("s")
    wid = s * NC + c
    base = wid * R

    def idx_start(n, b):
        pltpu.async_copy(dst_hbm.at[pl.ds((base + n) * CHUNK, CHUNK)],
                         dstb[b], semi[b])

    def idx_wait(n, b):
        pltpu.make_async_copy(dst_hbm.at[pl.ds((base + n) * CHUNK, CHUNK)],
                              dstb[b], semi[b]).wait()

    for i in range(CHUNK // 16):
        ones[pl.ds(i * 16, 16)] = jnp.full((16,), 1.0, _f32)
    for i in range(RPT // 16):
        stg[pl.ds(i * 16, 16)] = jnp.zeros((16,), _f32)
    pltpu.sync_copy(stg, acc.at[pl.ds(s * RPT, RPT)])
    idx_start(0, 0)
    idx_start(1, 1)
    plsc.subcore_barrier()

    def step(n, b):
        idx_wait(n, b)
        pltpu.sync_copy(ones, acc.at[dstb[b]], add=True)

        @pl.when(n + 2 < R)
        def _():
            idx_start(n + 2, b)

    def body(q, carry):
        step(2 * q, 0)
        step(2 * q + 1, 1)
        return carry

    lax.fori_loop(0, R // 2, body, 0, unroll=False)
    plsc.subcore_barrier()
    pltpu.sync_copy(acc.at[pl.ds(s * RPT, RPT)], stg)
    pltpu.sync_copy(stg, out_hbm.at[pl.ds(c * NP + s * RPT, RPT)])


_sc_deg = functools.partial(
    pl.kernel,
    out_type=jax.ShapeDtypeStruct((NC * NP,), _f32),
    mesh=_mesh,
    scratch_types=[
        [pltpu.VMEM((CHUNK,), jnp.int32) for _ in range(2)],
        pltpu.VMEM((CHUNK,), _f32),
        pltpu.VMEM((RPT,), _f32),
        pltpu.VMEM_SHARED((NP,), _f32),
        [pltpu.SemaphoreType.DMA for _ in range(2)],
    ],
)(_sc_deg_body)


def _tc_first(x_ref, w_ref, deg_ref, g_ref, dinv_ref):
    dsum = deg_ref[0] + deg_ref[1] + 1.0
    dinv_bc = jnp.broadcast_to(lax.rsqrt(dsum), (N, D))
    h = jnp.dot(x_ref[...], w_ref[...],
                preferred_element_type=_f32,
                precision=lax.Precision.HIGHEST)
    g_ref[...] = h * dinv_bc
    dinv_ref[...] = dinv_bc


def _tc_mid(s_ref, g_ref, dinv_ref, b_ref, w_ref, o_ref):
    t = (s_ref[0] + s_ref[1] + g_ref[...]) * dinv_ref[...] + b_ref[...]
    t = jnp.maximum(t, 0.0)
    t = t - jnp.mean(t, axis=0, keepdims=True)
    t = t * lax.rsqrt(1e-5 + jnp.sum(t * t) / N)
    h = jnp.dot(t, w_ref[...],
                preferred_element_type=_f32,
                precision=lax.Precision.HIGHEST)
    o_ref[...] = h * dinv_ref[...]


def _tc_last(s_ref, g_ref, dinv_ref, b_ref, o_ref):
    t = (s_ref[0] + s_ref[1] + g_ref[...]) * dinv_ref[...] + b_ref[...]
    o_ref[...] = jnp.maximum(t, 0.0)


_tc_first_call = pl.pallas_call(
    _tc_first,
    out_shape=[jax.ShapeDtypeStruct((N, D), _f32),
               jax.ShapeDtypeStruct((N, D), _f32)],
)
_tc_mid_call = pl.pallas_call(
    _tc_mid, out_shape=jax.ShapeDtypeStruct((N, D), _f32))
_tc_last_call = pl.pallas_call(
    _tc_last, out_shape=jax.ShapeDtypeStruct((N, D), _f32))


def kernel(x, edge_index, W0, b0, W1, b1, W2, b2):
    npad = EPAD - E
    pad_src = jnp.zeros((npad,), jnp.int32)
    pad_dst = N + (jnp.arange(npad, dtype=jnp.int32) % (NP - N))
    src1 = jnp.concatenate([edge_index[0], pad_src])
    dst1 = jnp.concatenate([edge_index[1], pad_dst])

    degp = _sc_deg(dst1).reshape(NC, NP)[:, :N]
    deg3 = degp.reshape(NC, N, 1)
    g0, dinv_bc = _tc_first_call(x, W0, deg3)
    s = _sc_scatter(g0, src1, dst1)[:, :N]
    g1 = _tc_mid_call(s, g0, dinv_bc, b0.reshape(1, D), W1)
    s = _sc_scatter(g1, src1, dst1)[:, :N]
    g2 = _tc_mid_call(s, g1, dinv_bc, b1.reshape(1, D), W2)
    s = _sc_scatter(g2, src1, dst1)[:, :N]
    return _tc_last_call(s, g2, dinv_bc, b2.reshape(1, D))
